# baseline (speedup 1.0000x reference)
import jax
import jax.numpy as jnp
from jax import lax
from jax.experimental import pallas as pl
from jax.experimental.pallas import tpu as pltpu
from jax.experimental.pallas import tpu_sc as plsc

F = 26
CARD = 100000
D = 32
B = 16384
NC, NS, L = 2, 16, 16
NW = NC * NS
RPW = B // NW
G = 128
NG = RPW // G


def _stage_idx(xT, idx, f, base, isem):
    return [
        pltpu.async_copy(xT.at[f, pl.ds(base + g * G, G)], idx.at[g], isem)
        for g in range(NG)
    ]


def _body(xT, table, bias, out, idx0, idx1, rows0, rows1, rt0, rt1,
          bias_v, gsem, isem, osem):
    wid = lax.axis_index("s") * NC + lax.axis_index("c")
    base = wid * RPW
    pltpu.sync_copy(bias, bias_v)

    idxs = (idx0, idx1)
    rows = (rows0, rows1)
    rts = (rt0, rt1)
    row_lo = lax.iota(jnp.int32, L)
    row_hi = row_lo + L
    zeros = jnp.zeros((L,), jnp.int32)

    def prep_and_fire(f, p):
        idx, buf = idxs[p], rows[p]
        off = f * CARD
        for g in range(NG):
            for k in range(G // L):
                idx[g, pl.ds(k * L, L)] = idx[g, pl.ds(k * L, L)] + off
        for g in range(NG):
            pltpu.async_copy(table.at[idx.at[g]],
                             buf.at[pl.ds(g * G, G)], gsem)

    for cp in _stage_idx(xT, idxs[0], 0, base, isem):
        cp.wait()
    prep_and_fire(0, 0)
    icps_holder = [_stage_idx(xT, idxs[1], 1, base, isem)]

    def run_feature(f, p, fire_next, stage_next2):
        rbuf, tbuf = rows[p], rts[p]
        for g in range(NG):
            pltpu.make_async_copy(
                table.at[pl.ds(0, G)], rbuf.at[pl.ds(g * G, G)], gsem).wait()
        if fire_next:
            for cp in icps_holder[0]:
                cp.wait()
            prep_and_fire(f + 1, 1 - p)
        if stage_next2:
            icps_holder[0] = _stage_idx(xT, idxs[p], f + 2, base, isem)
        if f >= 2:
            pltpu.make_async_copy(
                rts[p], out.at[0, pl.ds(0, D), pl.ds(0, RPW)], osem).wait()
        b_lo = bias_v[f, pl.ds(0, L)]
        b_hi = bias_v[f, pl.ds(L, L)]

        @plsc.parallel_loop(0, RPW, 1, unroll=4, carry=zeros)
        def _(i, col):
            v0 = rbuf[i, pl.ds(0, L)] + b_lo
            v1 = rbuf[i, pl.ds(L, L)] + b_hi
            plsc.store_scatter(tbuf, [row_lo, col], v0)
            plsc.store_scatter(tbuf, [row_hi, col], v1)
            return col + 1

        pltpu.async_copy(rts[p], out.at[f, pl.ds(0, D), pl.ds(base, RPW)],
                         osem)

    for f in range(F):
        run_feature(f, f % 2, fire_next=(f + 1 < F), stage_next2=(f + 2 < F))
    for p in range(2):
        pltpu.make_async_copy(
            rts[p], out.at[0, pl.ds(0, D), pl.ds(0, RPW)], osem).wait()


def kernel(x, table, bias):
    xT = x.T
    mesh = plsc.VectorSubcoreMesh(core_axis_name="c", subcore_axis_name="s")
    k = pl.kernel(
        _body,
        out_type=jax.ShapeDtypeStruct((F, D, B), jnp.float32),
        mesh=mesh,
        scratch_types=[
            pltpu.VMEM((NG, G), jnp.int32),
            pltpu.VMEM((NG, G), jnp.int32),
            pltpu.VMEM((RPW, D), jnp.float32),
            pltpu.VMEM((RPW, D), jnp.float32),
            pltpu.VMEM((D, RPW), jnp.float32),
            pltpu.VMEM((D, RPW), jnp.float32),
            pltpu.VMEM((F, D), jnp.float32),
            pltpu.SemaphoreType.DMA,
            pltpu.SemaphoreType.DMA,
            pltpu.SemaphoreType.DMA,
        ],
        compiler_params=pltpu.CompilerParams(
            use_tc_tiling_on_sc=False, needs_layout_passes=False),
    )
    outP = k(xT, table, bias)
    return outP.transpose(2, 0, 1)
